# baseline (speedup 1.0000x reference)
import functools

import jax
import jax.numpy as jnp
from jax import lax
from jax.experimental import pallas as pl
from jax.experimental.pallas import tpu as pltpu
from jax.experimental.pallas import tpu_sc as plsc

D = 1024
_info = plsc.get_sparse_core_info()
_NC, _NS = _info.num_cores, _info.num_subcores
_NW = _NC * _NS

_CH = 64


def _sc_body(seg_hbm, table_hbm, out_hbm, idx_v, buf, sem):
    b_per_w = seg_hbm.shape[0] // _NW
    wid = lax.axis_index("s") * _NC + lax.axis_index("c")
    base = wid * b_per_w
    pltpu.sync_copy(seg_hbm.at[pl.ds(base, b_per_w)], idx_v)
    for i in range(b_per_w // _CH):
        pltpu.async_copy(
            table_hbm.at[idx_v.at[pl.ds(i * _CH, _CH)]], buf, sem
        ).wait()
        pltpu.sync_copy(buf, out_hbm.at[pl.ds(base + i * _CH, _CH)])


@jax.jit
def _sc_lookup(seg_flat, table):
    b = seg_flat.shape[0]
    b_per_w = b // _NW
    mesh = plsc.VectorSubcoreMesh(core_axis_name="c", subcore_axis_name="s")
    return pl.kernel(
        _sc_body,
        out_type=jax.ShapeDtypeStruct((b, D), jnp.float32),
        mesh=mesh,
        scratch_types=[
            pltpu.VMEM((b_per_w,), jnp.int32),
            pltpu.VMEM((_CH, D), jnp.float32),
            pltpu.SemaphoreType.DMA,
        ],
    )(seg_flat, table)


def kernel(segments, table):
    bsz, seq = segments.shape
    seg_flat = segments.reshape(bsz * seq).astype(jnp.int32)
    out = _sc_lookup(seg_flat, table)
    return out.reshape(bsz, seq, D)
